# baseline (speedup 1.0000x reference)
import jax
import jax.numpy as jnp
from jax.experimental import pallas as pl

NUM_NODES_K = 50000
EMBED_DIM_K = 128
BATCH_K = 8
BLOCK_N = 10000


def _bcast_kernel(t_ref, o_ref):
    o_ref[0, :, :] = t_ref[...]


def kernel(table, batch_size):
    del batch_size
    grid = (NUM_NODES_K // BLOCK_N, BATCH_K)
    out = pl.pallas_call(
        _bcast_kernel,
        grid=grid,
        in_specs=[pl.BlockSpec((BLOCK_N, EMBED_DIM_K), lambda i, b: (i, 0))],
        out_specs=pl.BlockSpec((1, BLOCK_N, EMBED_DIM_K),
                               lambda i, b: (b, i, 0)),
        out_shape=jax.ShapeDtypeStruct((BATCH_K, NUM_NODES_K, EMBED_DIM_K),
                                       table.dtype),
    )(table)
    return out


# speedup vs baseline: 1.1435x; 1.1435x over previous
import jax
import jax.numpy as jnp
from jax.experimental import pallas as pl
from jax.experimental.pallas import tpu as pltpu

NUM_NODES_K = 50000
EMBED_DIM_K = 128
BATCH_K = 8
N_CHUNKS = 10
CHUNK_N = NUM_NODES_K // N_CHUNKS


def _bcast_kernel(t_hbm, o_hbm, buf, in_sems, out_sems):
    in_copies = []
    for c in range(N_CHUNKS):
        rows = pl.ds(c * CHUNK_N, CHUNK_N)
        cp = pltpu.make_async_copy(t_hbm.at[rows, :], buf.at[rows, :],
                                   in_sems.at[c])
        cp.start()
        in_copies.append(cp)
    out_copies = []
    for c in range(N_CHUNKS):
        in_copies[c].wait()
        rows = pl.ds(c * CHUNK_N, CHUNK_N)
        for b in range(BATCH_K):
            cp = pltpu.make_async_copy(buf.at[rows, :],
                                       o_hbm.at[b, rows, :],
                                       out_sems.at[b])
            cp.start()
            out_copies.append(cp)
    for cp in out_copies:
        cp.wait()


def kernel(table, batch_size):
    del batch_size
    out = pl.pallas_call(
        _bcast_kernel,
        in_specs=[pl.BlockSpec(memory_space=pl.ANY)],
        out_specs=pl.BlockSpec(memory_space=pl.ANY),
        out_shape=jax.ShapeDtypeStruct((BATCH_K, NUM_NODES_K, EMBED_DIM_K),
                                       table.dtype),
        scratch_shapes=[
            pltpu.VMEM((NUM_NODES_K, EMBED_DIM_K), jnp.float32),
            pltpu.SemaphoreType.DMA((N_CHUNKS,)),
            pltpu.SemaphoreType.DMA((BATCH_K,)),
        ],
    )(table)
    return out


# speedup vs baseline: 1.1446x; 1.0009x over previous
import jax
import jax.numpy as jnp
from jax.experimental import pallas as pl
from jax.experimental.pallas import tpu as pltpu

NUM_NODES_K = 50000
EMBED_DIM_K = 128
BATCH_K = 8
N_CHUNKS = 20
CHUNK_N = NUM_NODES_K // N_CHUNKS


def _bcast_kernel(t_hbm, o_hbm, buf, in_sems, out_sems):
    in_copies = []
    for c in range(N_CHUNKS):
        rows = pl.ds(c * CHUNK_N, CHUNK_N)
        cp = pltpu.make_async_copy(t_hbm.at[rows, :], buf.at[rows, :],
                                   in_sems.at[c])
        cp.start()
        in_copies.append(cp)
    out_copies = []
    for c in range(N_CHUNKS):
        in_copies[c].wait()
        rows = pl.ds(c * CHUNK_N, CHUNK_N)
        for b in range(BATCH_K):
            cp = pltpu.make_async_copy(buf.at[rows, :],
                                       o_hbm.at[b, rows, :],
                                       out_sems.at[b])
            cp.start()
            out_copies.append(cp)
    for cp in out_copies:
        cp.wait()


def kernel(table, batch_size):
    del batch_size
    out = pl.pallas_call(
        _bcast_kernel,
        in_specs=[pl.BlockSpec(memory_space=pl.ANY)],
        out_specs=pl.BlockSpec(memory_space=pl.ANY),
        out_shape=jax.ShapeDtypeStruct((BATCH_K, NUM_NODES_K, EMBED_DIM_K),
                                       table.dtype),
        scratch_shapes=[
            pltpu.VMEM((NUM_NODES_K, EMBED_DIM_K), jnp.float32),
            pltpu.SemaphoreType.DMA((N_CHUNKS,)),
            pltpu.SemaphoreType.DMA((BATCH_K,)),
        ],
    )(table)
    return out
